# baseline (speedup 1.0000x reference)
import functools

import jax
import jax.numpy as jnp
from jax import lax
from jax.experimental import pallas as pl
from jax.experimental.pallas import tpu as pltpu
from jax.experimental.pallas import tpu_sc as plsc

T = 2048
D = 1024
DFF = 2048
E = 8
K = 2
ALPHA = 0.001

B = 128
NB = (K * T) // B + E
NBB = NB * B
NSH = T // B
NBT = NB + NSH
NSEG = E + 3
LANES = 128
NEG = -1e30


def _gate_body(x_ref, wg_ref, bias_ref, probs_ref, ti_ref, tw_ref):
    s = lax.dot_general(x_ref[...], wg_ref[...], (((1,), (1,)), ((), ())),
                        preferred_element_type=jnp.float32)
    col = lax.broadcasted_iota(jnp.int32, s.shape, 1)
    valid = col < E
    s = jnp.where(valid, s, NEG)
    m = jnp.max(s, axis=1, keepdims=True)
    p = jnp.exp(s - m)
    p = jnp.where(valid, p, 0.0)
    probs = p / jnp.sum(p, axis=1, keepdims=True)
    biased = probs + bias_ref[...][0:1, :]
    biased = jnp.where(valid, biased, NEG)
    m1 = jnp.max(biased, axis=1, keepdims=True)
    i1 = jnp.min(jnp.where(biased == m1, col, LANES), axis=1, keepdims=True)
    b2 = jnp.where(col == i1, NEG, biased)
    m2 = jnp.max(b2, axis=1, keepdims=True)
    i2 = jnp.min(jnp.where(b2 == m2, col, LANES), axis=1, keepdims=True)
    w1 = jnp.sum(jnp.where(col == i1, probs, 0.0), axis=1, keepdims=True)
    w2 = jnp.sum(jnp.where(col == i2, probs, 0.0), axis=1, keepdims=True)
    colE = col[:, :E]
    probs_ref[...] = probs[:, :E]
    ti_ref[...] = jnp.where(colE == 0, i1, jnp.where(colE == 1, i2, 0))
    tw_ref[...] = jnp.where(colE == 0, w1, jnp.where(colE == 1, w2, 0.0))


def _gate(x, Wg, gate_bias):
    bt = 256
    wg_pad = jnp.zeros((LANES, D), jnp.float32).at[:E].set(Wg)
    bias_pad = jnp.zeros((8, LANES), jnp.float32).at[0, :E].set(gate_bias)
    return pl.pallas_call(
        _gate_body,
        grid=(T // bt,),
        in_specs=[
            pl.BlockSpec((bt, D), lambda i: (i, 0)),
            pl.BlockSpec((LANES, D), lambda i: (0, 0)),
            pl.BlockSpec((8, LANES), lambda i: (0, 0)),
        ],
        out_specs=[
            pl.BlockSpec((bt, E), lambda i: (i, 0)),
            pl.BlockSpec((bt, E), lambda i: (i, 0)),
            pl.BlockSpec((bt, E), lambda i: (i, 0)),
        ],
        out_shape=[
            jax.ShapeDtypeStruct((T, E), jnp.float32),
            jax.ShapeDtypeStruct((T, E), jnp.int32),
            jax.ShapeDtypeStruct((T, E), jnp.float32),
        ],
    )(x, wg_pad, bias_pad)


def _make_sc_gather(n_rows, chunk):
    info = plsc.get_sparse_core_info()
    nw = info.num_cores * info.num_subcores
    per_w = n_rows // nw
    assert n_rows % nw == 0 and per_w % chunk == 0 and chunk % 8 == 0
    n_iter = per_w // chunk
    nbuf = min(2, n_iter)
    mesh = plsc.VectorSubcoreMesh(core_axis_name="c", subcore_axis_name="s")

    @functools.partial(
        pl.kernel, mesh=mesh,
        out_type=jax.ShapeDtypeStruct((n_rows, D), jnp.float32),
        scratch_types=[
            pltpu.VMEM((per_w,), jnp.int32),
            pltpu.VMEM((nbuf, chunk, D), jnp.float32),
            pltpu.SemaphoreType.DMA,
            pltpu.SemaphoreType.DMA,
            pltpu.SemaphoreType.DMA,
            pltpu.SemaphoreType.DMA,
        ],
    )
    def k(table_hbm, idx_hbm, out_hbm, idx_v, rows_v, gs0, gs1, ss0, ss1):
        wid = lax.axis_index("s") * info.num_cores + lax.axis_index("c")
        base = wid * per_w
        pltpu.sync_copy(idx_hbm.at[pl.ds(base, per_w)], idx_v)
        gsems = [gs0, gs1]
        ssems = [ss0, ss1]
        gat = [None, None]
        scat = [None, None]
        for i in range(n_iter + 1):
            b = i % nbuf
            if i < n_iter:
                if scat[b] is not None:
                    scat[b].wait()
                    scat[b] = None
                gat[b] = pltpu.async_copy(
                    table_hbm.at[idx_v.at[pl.ds(i * chunk, chunk)]],
                    rows_v.at[b], gsems[b])
            if i >= 1:
                pb = (i - 1) % nbuf
                gat[pb].wait()
                scat[pb] = pltpu.async_copy(
                    rows_v.at[pb],
                    out_hbm.at[pl.ds(base + (i - 1) * chunk, chunk)],
                    ssems[pb])
        for b in range(nbuf):
            if scat[b] is not None:
                scat[b].wait()

    return k


def _moe_body(be_ref, seg_ref, ff_ref, se_ref, ns_ref,
              xs_ref, x_ref, bfc_ref, bproj_ref,
              wfc_hbm, wsfc_hbm, wproj_hbm, wsproj_hbm,
              out_ref, wfc_buf, wproj_buf, sems):
    i = pl.program_id(0)
    s = seg_ref[i]
    even = jax.lax.rem(s, 2) == 0
    first = ff_ref[i] == 1

    def fetch(seg_idx, slot):
        e = se_ref[seg_idx]

        @pl.when(e == E)
        def _():
            pltpu.make_async_copy(wsfc_hbm, wfc_buf.at[slot], sems.at[slot]
                                  ).start()
            pltpu.make_async_copy(wsproj_hbm, wproj_buf.at[slot],
                                  sems.at[slot]).start()

        @pl.when(e < E)
        def _():
            pltpu.make_async_copy(wfc_hbm.at[e], wfc_buf.at[slot],
                                  sems.at[slot]).start()
            pltpu.make_async_copy(wproj_hbm.at[e], wproj_buf.at[slot],
                                  sems.at[slot]).start()

    @pl.when(i == 0)
    def _():
        fetch(0, 0)

    def seg_start(slot):
        pltpu.make_async_copy(wsfc_hbm, wfc_buf.at[slot], sems.at[slot]).wait()
        pltpu.make_async_copy(wsproj_hbm, wproj_buf.at[slot],
                              sems.at[slot]).wait()

        @pl.when(s + 1 < ns_ref[0])
        def _():
            fetch(s + 1, 1 - slot)

    @pl.when(first & even)
    def _():
        seg_start(0)

    @pl.when(first & jnp.logical_not(even))
    def _():
        seg_start(1)

    xin = jnp.where(i < NB, xs_ref[...], x_ref[...])

    def compute(slot):
        h = lax.dot_general(xin, wfc_buf[slot], (((1,), (1,)), ((), ())),
                            preferred_element_type=jnp.float32)
        h = h + bfc_ref[...][0]
        h = h * jax.nn.sigmoid(h)
        o = lax.dot_general(h, wproj_buf[slot], (((1,), (1,)), ((), ())),
                            preferred_element_type=jnp.float32)
        out_ref[...] = o + bproj_ref[...][0]

    @pl.when(even)
    def _():
        compute(0)

    @pl.when(jnp.logical_not(even))
    def _():
        compute(1)


def _moe_mlp(xs, x, Wfc, bfc, Wproj, bproj, Ws_fc, bs_fc, Ws_proj, bs_proj,
             be_ext, seg_of_block, first_flag, seg_expert, nseg_arr):
    bfc_all = jnp.concatenate([bfc, bs_fc[None]], axis=0)[:, None, :]
    bproj_all = jnp.concatenate([bproj, bs_proj[None]], axis=0)[:, None, :]
    grid_spec = pltpu.PrefetchScalarGridSpec(
        num_scalar_prefetch=5,
        grid=(NBT,),
        in_specs=[
            pl.BlockSpec((B, D), lambda i, *refs: (jnp.minimum(i, NB - 1), 0)),
            pl.BlockSpec((B, D), lambda i, *refs: (jnp.maximum(i - NB, 0), 0)),
            pl.BlockSpec((1, 1, DFF), lambda i, be, *refs: (be[i], 0, 0)),
            pl.BlockSpec((1, 1, D), lambda i, be, *refs: (be[i], 0, 0)),
            pl.BlockSpec(memory_space=pl.ANY),
            pl.BlockSpec(memory_space=pl.ANY),
            pl.BlockSpec(memory_space=pl.ANY),
            pl.BlockSpec(memory_space=pl.ANY),
        ],
        out_specs=pl.BlockSpec((B, D), lambda i, *refs: (i, 0)),
        scratch_shapes=[
            pltpu.VMEM((2, DFF, D), jnp.float32),
            pltpu.VMEM((2, D, DFF), jnp.float32),
            pltpu.SemaphoreType.DMA((2,)),
        ],
    )
    return pl.pallas_call(
        _moe_body,
        grid_spec=grid_spec,
        out_shape=jax.ShapeDtypeStruct((NBT * B, D), jnp.float32),
        compiler_params=pltpu.CompilerParams(
            dimension_semantics=("arbitrary",),
            vmem_limit_bytes=100 * 1024 * 1024),
    )(be_ext, seg_of_block, first_flag, seg_expert, nseg_arr,
      xs, x, bfc_all, bproj_all, Wfc, Ws_fc, Wproj, Ws_proj)


def _add3_body(z_ref, g0_ref, g1_ref, tw_ref, y_ref):
    w = tw_ref[...]
    y_ref[...] = (z_ref[...] + w[:, 0:1] * g0_ref[...]
                  + w[:, 1:2] * g1_ref[...])


def _add3(contrib, g0, g1, tw):
    bt = 512
    off = NBB // bt
    return pl.pallas_call(
        _add3_body,
        grid=(T // bt,),
        in_specs=[pl.BlockSpec((bt, D), lambda i: (i + off, 0)),
                  pl.BlockSpec((bt, D), lambda i: (i, 0)),
                  pl.BlockSpec((bt, D), lambda i: (i, 0)),
                  pl.BlockSpec((bt, E), lambda i: (i, 0))],
        out_specs=pl.BlockSpec((bt, D), lambda i: (i, 0)),
        out_shape=jax.ShapeDtypeStruct((T, D), jnp.float32),
    )(contrib, g0, g1, tw)


def kernel(x, Wg, gate_bias, Wfc, bfc, Wproj, bproj, Ws_fc, bs_fc, Ws_proj, bs_proj):
    probs, ti, tw = _gate(x, Wg, gate_bias)
    i1, i2 = ti[:, 0], ti[:, 1]
    w1, w2 = tw[:, 0], tw[:, 1]

    e_f = jnp.stack([i1, i2], axis=1).reshape(-1)
    oh = (e_f[:, None] == jnp.arange(E)[None, :]).astype(jnp.int32)
    csum = jnp.cumsum(oh, axis=0)
    counts = csum[-1]
    rank = jnp.take_along_axis(csum, e_f[:, None], axis=1)[:, 0] - 1
    nb_e = (counts + B - 1) // B
    blk_start = jnp.cumsum(nb_e) - nb_e
    p = blk_start[e_f] * B + rank
    tok = jnp.repeat(jnp.arange(T, dtype=jnp.int32), K)
    src = (jnp.arange(NBB, dtype=jnp.int32) % T).at[p].set(
        tok, unique_indices=True)
    bidx = jnp.arange(NB)
    be = jnp.sum((bidx[:, None] >= blk_start[None, :]).astype(jnp.int32),
                 axis=1) - 1
    be = jnp.clip(be, 0, E - 1).astype(jnp.int32)
    pos0, pos1 = p[0::K], p[1::K]

    be_ext = jnp.concatenate([be, jnp.full((NSH,), E, jnp.int32)])
    prev = jnp.concatenate([jnp.array([-1], jnp.int32), be_ext[:-1]])
    first_flag = (be_ext != prev).astype(jnp.int32)
    seg_of_block = jnp.cumsum(first_flag) - 1
    nseg_arr = seg_of_block[-1:] + 1
    seg_expert = jnp.full((NSEG,), E, jnp.int32).at[seg_of_block].set(be_ext)

    xs = _make_sc_gather(NBB, 40)(x, src)
    contrib = _moe_mlp(xs, x, Wfc, bfc, Wproj, bproj,
                       Ws_fc, bs_fc, Ws_proj, bs_proj,
                       be_ext, seg_of_block, first_flag, seg_expert, nseg_arr)
    g0 = _make_sc_gather(T, 64)(contrib, pos0.astype(jnp.int32))
    g1 = _make_sc_gather(T, 64)(contrib, pos1.astype(jnp.int32))
    y = _add3(contrib, g0, g1, tw)

    expert_probs = probs.mean(axis=0)
    f_i = counts.astype(jnp.float32) * E / (K * T + 1e-06)
    load_balance_loss = ALPHA * jnp.sum(f_i * expert_probs)
    return (y, load_balance_loss)


# speedup vs baseline: 1.2997x; 1.2997x over previous
import functools

import jax
import jax.numpy as jnp
from jax import lax
from jax.experimental import pallas as pl
from jax.experimental.pallas import tpu as pltpu
from jax.experimental.pallas import tpu_sc as plsc

T = 2048
D = 1024
DFF = 2048
E = 8
K = 2
ALPHA = 0.001

B = 256
NB = (K * T) // B + E
NBB = NB * B
LANES = 128
NEG = -1e30


def _gate_body(x_ref, wg_ref, bias_ref, probs_ref, ti_ref, tw_ref):
    s = lax.dot_general(x_ref[...], wg_ref[...], (((1,), (1,)), ((), ())),
                        preferred_element_type=jnp.float32)
    col = lax.broadcasted_iota(jnp.int32, s.shape, 1)
    valid = col < E
    s = jnp.where(valid, s, NEG)
    m = jnp.max(s, axis=1, keepdims=True)
    p = jnp.exp(s - m)
    p = jnp.where(valid, p, 0.0)
    probs = p / jnp.sum(p, axis=1, keepdims=True)
    biased = probs + bias_ref[...][0:1, :]
    biased = jnp.where(valid, biased, NEG)
    m1 = jnp.max(biased, axis=1, keepdims=True)
    i1 = jnp.min(jnp.where(biased == m1, col, LANES), axis=1, keepdims=True)
    b2 = jnp.where(col == i1, NEG, biased)
    m2 = jnp.max(b2, axis=1, keepdims=True)
    i2 = jnp.min(jnp.where(b2 == m2, col, LANES), axis=1, keepdims=True)
    w1 = jnp.sum(jnp.where(col == i1, probs, 0.0), axis=1, keepdims=True)
    w2 = jnp.sum(jnp.where(col == i2, probs, 0.0), axis=1, keepdims=True)
    colE = col[:, :E]
    probs_ref[...] = probs[:, :E]
    ti_ref[...] = jnp.where(colE == 0, i1, jnp.where(colE == 1, i2, 0))
    tw_ref[...] = jnp.where(colE == 0, w1, jnp.where(colE == 1, w2, 0.0))


def _gate(x, Wg, gate_bias):
    bt = 256
    wg_pad = jnp.zeros((LANES, D), jnp.float32).at[:E].set(Wg)
    bias_pad = jnp.zeros((8, LANES), jnp.float32).at[0, :E].set(gate_bias)
    return pl.pallas_call(
        _gate_body,
        grid=(T // bt,),
        in_specs=[
            pl.BlockSpec((bt, D), lambda i: (i, 0)),
            pl.BlockSpec((LANES, D), lambda i: (0, 0)),
            pl.BlockSpec((8, LANES), lambda i: (0, 0)),
        ],
        out_specs=[
            pl.BlockSpec((bt, E), lambda i: (i, 0)),
            pl.BlockSpec((bt, E), lambda i: (i, 0)),
            pl.BlockSpec((bt, E), lambda i: (i, 0)),
        ],
        out_shape=[
            jax.ShapeDtypeStruct((T, E), jnp.float32),
            jax.ShapeDtypeStruct((T, E), jnp.int32),
            jax.ShapeDtypeStruct((T, E), jnp.float32),
        ],
    )(x, wg_pad, bias_pad)


def _make_sc_gather(n_rows, chunk):
    info = plsc.get_sparse_core_info()
    nw = info.num_cores * info.num_subcores
    per_w = n_rows // nw
    assert n_rows % nw == 0 and per_w % chunk == 0 and chunk % 8 == 0
    n_iter = per_w // chunk
    nbuf = min(2, n_iter)
    mesh = plsc.VectorSubcoreMesh(core_axis_name="c", subcore_axis_name="s")

    @functools.partial(
        pl.kernel, mesh=mesh,
        out_type=jax.ShapeDtypeStruct((n_rows, D), jnp.float32),
        scratch_types=[
            pltpu.VMEM((per_w,), jnp.int32),
            pltpu.VMEM((nbuf, chunk, D), jnp.float32),
            pltpu.SemaphoreType.DMA,
            pltpu.SemaphoreType.DMA,
            pltpu.SemaphoreType.DMA,
            pltpu.SemaphoreType.DMA,
        ],
    )
    def k(table_hbm, idx_hbm, out_hbm, idx_v, rows_v, gs0, gs1, ss0, ss1):
        wid = lax.axis_index("s") * info.num_cores + lax.axis_index("c")
        base = wid * per_w
        pltpu.sync_copy(idx_hbm.at[pl.ds(base, per_w)], idx_v)
        gsems = [gs0, gs1]
        ssems = [ss0, ss1]
        gat = [None, None]
        scat = [None, None]
        for i in range(n_iter + 1):
            b = i % nbuf
            if i < n_iter:
                if scat[b] is not None:
                    scat[b].wait()
                    scat[b] = None
                gat[b] = pltpu.async_copy(
                    table_hbm.at[idx_v.at[pl.ds(i * chunk, chunk)]],
                    rows_v.at[b], gsems[b])
            if i >= 1:
                pb = (i - 1) % nbuf
                gat[pb].wait()
                scat[pb] = pltpu.async_copy(
                    rows_v.at[pb],
                    out_hbm.at[pl.ds(base + (i - 1) * chunk, chunk)],
                    ssems[pb])
        for b in range(nbuf):
            if scat[b] is not None:
                scat[b].wait()

    return k


def _routed_body(be_ref, xs_ref, wfc_ref, bfc_ref, wproj_ref, bproj_ref,
                 out_ref):
    h = lax.dot_general(xs_ref[...], wfc_ref[...][0], (((1,), (1,)), ((), ())),
                        preferred_element_type=jnp.float32)
    h = h + bfc_ref[...][0]
    h = h * jax.nn.sigmoid(h)
    o = lax.dot_general(h, wproj_ref[...][0], (((1,), (1,)), ((), ())),
                        preferred_element_type=jnp.float32)
    out_ref[...] = o + bproj_ref[...][0]


def _routed_mlp(xs, Wfc, bfc, Wproj, bproj, be):
    grid_spec = pltpu.PrefetchScalarGridSpec(
        num_scalar_prefetch=1,
        grid=(NB,),
        in_specs=[
            pl.BlockSpec((B, D), lambda i, be: (i, 0)),
            pl.BlockSpec((1, DFF, D), lambda i, be: (be[i], 0, 0)),
            pl.BlockSpec((1, 1, DFF), lambda i, be: (be[i], 0, 0)),
            pl.BlockSpec((1, D, DFF), lambda i, be: (be[i], 0, 0)),
            pl.BlockSpec((1, 1, D), lambda i, be: (be[i], 0, 0)),
        ],
        out_specs=pl.BlockSpec((B, D), lambda i, be: (i, 0)),
    )
    return pl.pallas_call(
        _routed_body,
        grid_spec=grid_spec,
        out_shape=jax.ShapeDtypeStruct((NBB, D), jnp.float32),
        compiler_params=pltpu.CompilerParams(
            dimension_semantics=("arbitrary",),
            vmem_limit_bytes=100 * 1024 * 1024),
    )(be, xs, Wfc, bfc[:, None, :], Wproj, bproj[:, None, :])


def _shared_body(x_ref, wsfc_ref, bsfc_ref, wsproj_ref, bsproj_ref, z_ref):
    h = lax.dot_general(x_ref[...], wsfc_ref[...], (((1,), (1,)), ((), ())),
                        preferred_element_type=jnp.float32)
    h = h + bsfc_ref[...][0][None, :]
    h = h * jax.nn.sigmoid(h)
    z = lax.dot_general(h, wsproj_ref[...], (((1,), (1,)), ((), ())),
                        preferred_element_type=jnp.float32)
    z_ref[...] = z + bsproj_ref[...][0][None, :]


def _shared_mlp(x, Ws_fc, bs_fc, Ws_proj, bs_proj):
    bt = 256
    return pl.pallas_call(
        _shared_body,
        grid=(T // bt,),
        in_specs=[
            pl.BlockSpec((bt, D), lambda i: (i, 0)),
            pl.BlockSpec((DFF, D), lambda i: (0, 0)),
            pl.BlockSpec((1, DFF), lambda i: (0, 0)),
            pl.BlockSpec((D, DFF), lambda i: (0, 0)),
            pl.BlockSpec((1, D), lambda i: (0, 0)),
        ],
        out_specs=pl.BlockSpec((bt, D), lambda i: (i, 0)),
        out_shape=jax.ShapeDtypeStruct((T, D), jnp.float32),
    )(x, Ws_fc, bs_fc[None, :], Ws_proj, bs_proj[None, :])


def _add3_body(z_ref, g0_ref, g1_ref, tw_ref, y_ref):
    w = tw_ref[...]
    y_ref[...] = (z_ref[...] + w[:, 0:1] * g0_ref[...]
                  + w[:, 1:2] * g1_ref[...])


def _add3(z, g0, g1, tw):
    bt = 512
    return pl.pallas_call(
        _add3_body,
        grid=(T // bt,),
        in_specs=[pl.BlockSpec((bt, D), lambda i: (i, 0))] * 3
        + [pl.BlockSpec((bt, E), lambda i: (i, 0))],
        out_specs=pl.BlockSpec((bt, D), lambda i: (i, 0)),
        out_shape=jax.ShapeDtypeStruct((T, D), jnp.float32),
    )(z, g0, g1, tw)


def kernel(x, Wg, gate_bias, Wfc, bfc, Wproj, bproj, Ws_fc, bs_fc, Ws_proj, bs_proj):
    probs, ti, tw = _gate(x, Wg, gate_bias)
    i1, i2 = ti[:, 0], ti[:, 1]

    e_f = jnp.stack([i1, i2], axis=1).reshape(-1)
    oh = (e_f[:, None] == jnp.arange(E)[None, :]).astype(jnp.int32)
    csum = jnp.cumsum(oh, axis=0)
    counts = csum[-1]
    rank = jnp.take_along_axis(csum, e_f[:, None], axis=1)[:, 0] - 1
    nb_e = (counts + B - 1) // B
    blk_start = jnp.cumsum(nb_e) - nb_e
    p = blk_start[e_f] * B + rank
    tok = jnp.repeat(jnp.arange(T, dtype=jnp.int32), K)
    src = (jnp.arange(NBB, dtype=jnp.int32) % T).at[p].set(
        tok, unique_indices=True)
    bidx = jnp.arange(NB)
    be = jnp.sum((bidx[:, None] >= blk_start[None, :]).astype(jnp.int32),
                 axis=1) - 1
    be = jnp.clip(be, 0, E - 1).astype(jnp.int32)
    pos0, pos1 = p[0::K], p[1::K]

    z = _shared_mlp(x, Ws_fc, bs_fc, Ws_proj, bs_proj)
    xs = _make_sc_gather(NBB, 48)(x, src)
    contrib = _routed_mlp(xs, Wfc, bfc, Wproj, bproj, be)
    g0 = _make_sc_gather(T, 64)(contrib, pos0.astype(jnp.int32))
    g1 = _make_sc_gather(T, 64)(contrib, pos1.astype(jnp.int32))
    y = _add3(z, g0, g1, tw)

    expert_probs = probs.mean(axis=0)
    f_i = counts.astype(jnp.float32) * E / (K * T + 1e-06)
    load_balance_loss = ALPHA * jnp.sum(f_i * expert_probs)
    return (y, load_balance_loss)
